# baseline (speedup 1.0000x reference)
import functools

import jax
import jax.numpy as jnp
import numpy as np
from jax.experimental import pallas as pl
from jax.experimental.pallas import tpu as pltpu

ALPHA = 3.0
TOPK = 64
INT_MIN = np.int32(-2147483648)


def _rotl(x, r):
    return jnp.left_shift(x, r) | jax.lax.shift_right_logical(x, 32 - r)


def _threefry_bits(p):
    ks0 = np.int32(0)
    ks1 = np.int32(42)
    ks2 = np.int32((0 ^ 42 ^ 0x1BD11BDA) & 0xFFFFFFFF)
    rot_a = (13, 15, 26, 6)
    rot_b = (17, 29, 16, 24)

    x0 = jnp.zeros_like(p) + ks0
    x1 = p + ks1

    def rounds(x0, x1, rots):
        for r in rots:
            x0 = x0 + x1
            x1 = _rotl(x1, r)
            x1 = x0 ^ x1
        return x0, x1

    x0, x1 = rounds(x0, x1, rot_a)
    x0 = x0 + ks1
    x1 = x1 + np.int32(ks2 + 1)
    x0, x1 = rounds(x0, x1, rot_b)
    x0 = x0 + ks2
    x1 = x1 + np.int32(ks0 + 2)
    x0, x1 = rounds(x0, x1, rot_a)
    x0 = x0 + ks0
    x1 = x1 + np.int32(ks1 + 3)
    x0, x1 = rounds(x0, x1, rot_b)
    x0 = x0 + ks1
    x1 = x1 + np.int32(ks2 + 4)
    x0, x1 = rounds(x0, x1, rot_a)
    x0 = x0 + ks2
    x1 = x1 + np.int32(ks0 + 5)
    return x0 ^ x1


def _embed_kernel(e1, e2, w1, w2, b1r, b2r, nvcat, *, dim):
    f32 = jnp.float32
    dot = jax.lax.dot_general
    dn_nt = (((1,), (1,)), ((), ()))

    x1 = jnp.tanh(ALPHA * (dot(e1[...], w1[...], dn_nt, preferred_element_type=f32) + b1r[...]))
    x2 = jnp.tanh(ALPHA * (dot(e2[...], w2[...], dn_nt, preferred_element_type=f32) + b2r[...]))
    nvcat[:, :dim] = x1
    nvcat[:, dim:] = x2


def _embed_t_kernel(e1, e2, w1, w2, b1c, b2c, wbig, *, dim):
    f32 = jnp.float32
    dot = jax.lax.dot_general
    dn_nt = (((1,), (1,)), ((), ()))

    x1t = jnp.tanh(ALPHA * (dot(w1[...], e1[...], dn_nt, preferred_element_type=f32) + b1c[...]))
    wbig[dim:, :] = x1t
    x2t = jnp.tanh(ALPHA * (dot(w2[...], e2[...], dn_nt, preferred_element_type=f32) + b2c[...]))
    wbig[:dim, :] = x2t


def _adj_kernel(s10_ref, nvcat, wbig, out, pert_scr, pflat_scr, *, n, rblk, nbisect):
    f32 = jnp.float32
    i32 = jnp.int32
    g = pl.program_id(0)

    @pl.when(g == 0)
    def _():
        ii = jax.lax.broadcasted_iota(i32, (rblk, n), 0)
        jj = jax.lax.broadcasted_iota(i32, (rblk, n), 1)
        pflat_scr[...] = ii * n + jj

    z = nvcat[...]
    s = 0.5 * jax.lax.dot_general(
        z, wbig[...], (((1,), (0,)), ((), ())), preferred_element_type=f32)

    adj = jnp.maximum(s, 0.0) + jnp.log1p(jnp.exp(-jnp.abs(s))) - 0.5
    out[...] = adj

    p = pflat_scr[...] + g * (rblk * n)
    bits = _threefry_bits(p)
    fbits = jax.lax.shift_right_logical(bits, 9) | np.int32(0x3F800000)
    u = jax.lax.bitcast_convert_type(fbits, f32) - 1.0
    noise = -jnp.log(-jnp.log(u + 1e-10) + 1e-10)
    pert = adj + noise
    pert_scr[...] = pert

    hi = jnp.max(pert, axis=1, keepdims=True)
    lo = jnp.full((rblk, 1), -3.7, dtype=f32)
    for _ in range(nbisect):
        mid = 0.5 * (lo + hi)
        cnt = jnp.sum((pert_scr[...] >= mid).astype(f32), axis=1, keepdims=True)
        big = cnt >= float(TOPK)
        lo = jnp.where(big, mid, lo)
        hi = jnp.where(big, hi, mid)
    thr = lo

    s10 = s10_ref[0, 0]
    mask = 1.0 / (1.0 + jnp.exp(-(pert_scr[...] - thr) * s10))
    out[...] = jnp.maximum(out[...] * mask, 0.0)


def _pick_rblk(n):
    best = 8
    for r in range(8, 257, 8):
        if n % r == 0:
            best = r
    return best


@functools.partial(jax.jit, static_argnames=())
def kernel(idx, emb1, emb2, W1, b1, W2, b2, temperature):
    n, dim = emb1.shape
    f32 = jnp.float32

    e1 = jnp.take(emb1, idx, axis=0)
    e2 = jnp.take(emb2, idx, axis=0)

    nb = n // 10 if n % 10 == 0 else n
    nvcat = pl.pallas_call(
        functools.partial(_embed_kernel, dim=dim),
        grid=(n // nb,),
        in_specs=[
            pl.BlockSpec((nb, dim), lambda g: (g, 0)),
            pl.BlockSpec((nb, dim), lambda g: (g, 0)),
            pl.BlockSpec((dim, dim), lambda g: (0, 0)),
            pl.BlockSpec((dim, dim), lambda g: (0, 0)),
            pl.BlockSpec((1, dim), lambda g: (0, 0)),
            pl.BlockSpec((1, dim), lambda g: (0, 0)),
        ],
        out_specs=pl.BlockSpec((nb, 2 * dim), lambda g: (g, 0)),
        out_shape=jax.ShapeDtypeStruct((n, 2 * dim), f32),
    )(e1, e2, W1, W2, b1.reshape(1, dim), b2.reshape(1, dim))

    wbig = pl.pallas_call(
        functools.partial(_embed_t_kernel, dim=dim),
        in_specs=[
            pl.BlockSpec((n, dim), lambda: (0, 0)),
            pl.BlockSpec((n, dim), lambda: (0, 0)),
            pl.BlockSpec((dim, dim), lambda: (0, 0)),
            pl.BlockSpec((dim, dim), lambda: (0, 0)),
            pl.BlockSpec((dim, 1), lambda: (0, 0)),
            pl.BlockSpec((dim, 1), lambda: (0, 0)),
        ],
        out_specs=pl.BlockSpec((2 * dim, n), lambda: (0, 0)),
        out_shape=jax.ShapeDtypeStruct((2 * dim, n), f32),
    )(e1, e2, W1, W2, b1.reshape(dim, 1), b2.reshape(dim, 1))

    rblk = _pick_rblk(n)
    s10 = (10.0 / temperature.astype(f32)).reshape(1, 1)
    out = pl.pallas_call(
        functools.partial(_adj_kernel, n=n, rblk=rblk, nbisect=14),
        grid=(n // rblk,),
        in_specs=[
            pl.BlockSpec(memory_space=pltpu.SMEM),
            pl.BlockSpec((rblk, 2 * dim), lambda g: (g, 0)),
            pl.BlockSpec((2 * dim, n), lambda g: (0, 0)),
        ],
        out_specs=pl.BlockSpec((rblk, n), lambda g: (g, 0)),
        out_shape=jax.ShapeDtypeStruct((n, n), f32),
        scratch_shapes=[
            pltpu.VMEM((rblk, n), f32),
            pltpu.VMEM((rblk, n), jnp.int32),
        ],
    )(s10, nvcat, wbig)
    return out


# speedup vs baseline: 1.0886x; 1.0886x over previous
import functools

import jax
import jax.numpy as jnp
import numpy as np
from jax.experimental import pallas as pl
from jax.experimental.pallas import tpu as pltpu

ALPHA = 3.0
TOPK = 64
INT_MIN = np.int32(-2147483648)


def _rotl(x, r):
    return jnp.left_shift(x, r) | jax.lax.shift_right_logical(x, 32 - r)


def _threefry_bits(p):
    ks0 = np.int32(0)
    ks1 = np.int32(42)
    ks2 = np.int32((0 ^ 42 ^ 0x1BD11BDA) & 0xFFFFFFFF)
    rot_a = (13, 15, 26, 6)
    rot_b = (17, 29, 16, 24)

    x0 = jnp.zeros_like(p) + ks0
    x1 = p + ks1

    def rounds(x0, x1, rots):
        for r in rots:
            x0 = x0 + x1
            x1 = _rotl(x1, r)
            x1 = x0 ^ x1
        return x0, x1

    x0, x1 = rounds(x0, x1, rot_a)
    x0 = x0 + ks1
    x1 = x1 + np.int32(ks2 + 1)
    x0, x1 = rounds(x0, x1, rot_b)
    x0 = x0 + ks2
    x1 = x1 + np.int32(ks0 + 2)
    x0, x1 = rounds(x0, x1, rot_a)
    x0 = x0 + ks0
    x1 = x1 + np.int32(ks1 + 3)
    x0, x1 = rounds(x0, x1, rot_b)
    x0 = x0 + ks1
    x1 = x1 + np.int32(ks2 + 4)
    x0, x1 = rounds(x0, x1, rot_a)
    x0 = x0 + ks2
    x1 = x1 + np.int32(ks0 + 5)
    return x0 ^ x1


def _embed_kernel(e1, e2, w1, w2, b1r, b2r, nvcat, *, dim):
    f32 = jnp.float32
    dot = jax.lax.dot_general
    dn_nt = (((1,), (1,)), ((), ()))

    x1 = jnp.tanh(ALPHA * (dot(e1[...], w1[...], dn_nt, preferred_element_type=f32) + b1r[...]))
    x2 = jnp.tanh(ALPHA * (dot(e2[...], w2[...], dn_nt, preferred_element_type=f32) + b2r[...]))
    nvcat[:, :dim] = x1
    nvcat[:, dim:] = x2


def _embed_t_kernel(e1, e2, w1, w2, b1c, b2c, wbig, *, dim):
    f32 = jnp.float32
    dot = jax.lax.dot_general
    dn_nt = (((1,), (1,)), ((), ()))

    x1t = jnp.tanh(ALPHA * (dot(w1[...], e1[...], dn_nt, preferred_element_type=f32) + b1c[...]))
    wbig[dim:, :] = x1t
    x2t = jnp.tanh(ALPHA * (dot(w2[...], e2[...], dn_nt, preferred_element_type=f32) + b2c[...]))
    wbig[:dim, :] = x2t


def _adj_kernel(s10_ref, nvcat, wbig, out, pert_scr, pflat_scr, q_scr, *, n, rblk):
    f32 = jnp.float32
    i32 = jnp.int32
    g = pl.program_id(0)

    @pl.when(g == 0)
    def _():
        ii = jax.lax.broadcasted_iota(i32, (rblk, n), 0)
        jj = jax.lax.broadcasted_iota(i32, (rblk, n), 1)
        pflat_scr[...] = ii * n + jj

    z = nvcat[...]
    s = 0.5 * jax.lax.dot_general(
        z, wbig[...], (((1,), (0,)), ((), ())), preferred_element_type=f32)

    adj = jnp.maximum(s, 0.0) + jnp.log1p(jnp.exp(-jnp.abs(s))) - 0.5
    out[...] = adj

    p = pflat_scr[...] + g * (rblk * n)
    bits = _threefry_bits(p)
    fbits = jax.lax.shift_right_logical(bits, 9) | np.int32(0x3F800000)
    u = jax.lax.bitcast_convert_type(fbits, f32) - 1.0
    noise = -jnp.log(-jnp.log(u + 1e-10) + 1e-10)
    pert = adj + noise
    pert_scr[...] = pert

    i16 = jnp.int16
    lo0 = -3.7
    hi = jnp.max(pert, axis=1, keepdims=True)
    srow = 65535.0 / (hi - lo0)
    qf = (pert - lo0) * srow - 32768.0
    q_scr[...] = qf.astype(i32).astype(i16)

    def body(_, carry):
        t, step = carry
        cand = t + step
        cnt = jnp.sum(q_scr[...] >= cand.astype(i16), axis=1, keepdims=True,
                      dtype=f32)
        big = cnt >= float(TOPK)
        return jnp.where(big, cand, t), jax.lax.shift_right_logical(step, 1)

    t0 = jnp.full((rblk, 1), -32768, dtype=i32)
    s0 = jnp.full((rblk, 1), 32768, dtype=i32)
    t, _ = jax.lax.fori_loop(0, 16, body, (t0, s0))
    thr = (t.astype(f32) + 32768.0) / srow + lo0

    s10 = s10_ref[0, 0]
    mask = 1.0 / (1.0 + jnp.exp(-(pert_scr[...] - thr) * s10))
    out[...] = jnp.maximum(out[...] * mask, 0.0)


def _pick_rblk(n):
    best = 8
    for r in range(8, 257, 8):
        if n % r == 0:
            best = r
    return best


@functools.partial(jax.jit, static_argnames=())
def kernel(idx, emb1, emb2, W1, b1, W2, b2, temperature):
    n, dim = emb1.shape
    f32 = jnp.float32

    e1 = jnp.take(emb1, idx, axis=0)
    e2 = jnp.take(emb2, idx, axis=0)

    nb = n // 10 if n % 10 == 0 else n
    nvcat = pl.pallas_call(
        functools.partial(_embed_kernel, dim=dim),
        grid=(n // nb,),
        in_specs=[
            pl.BlockSpec((nb, dim), lambda g: (g, 0)),
            pl.BlockSpec((nb, dim), lambda g: (g, 0)),
            pl.BlockSpec((dim, dim), lambda g: (0, 0)),
            pl.BlockSpec((dim, dim), lambda g: (0, 0)),
            pl.BlockSpec((1, dim), lambda g: (0, 0)),
            pl.BlockSpec((1, dim), lambda g: (0, 0)),
        ],
        out_specs=pl.BlockSpec((nb, 2 * dim), lambda g: (g, 0)),
        out_shape=jax.ShapeDtypeStruct((n, 2 * dim), f32),
    )(e1, e2, W1, W2, b1.reshape(1, dim), b2.reshape(1, dim))

    wbig = pl.pallas_call(
        functools.partial(_embed_t_kernel, dim=dim),
        in_specs=[
            pl.BlockSpec((n, dim), lambda: (0, 0)),
            pl.BlockSpec((n, dim), lambda: (0, 0)),
            pl.BlockSpec((dim, dim), lambda: (0, 0)),
            pl.BlockSpec((dim, dim), lambda: (0, 0)),
            pl.BlockSpec((dim, 1), lambda: (0, 0)),
            pl.BlockSpec((dim, 1), lambda: (0, 0)),
        ],
        out_specs=pl.BlockSpec((2 * dim, n), lambda: (0, 0)),
        out_shape=jax.ShapeDtypeStruct((2 * dim, n), f32),
    )(e1, e2, W1, W2, b1.reshape(dim, 1), b2.reshape(dim, 1))

    rblk = _pick_rblk(n)
    s10 = (10.0 / temperature.astype(f32)).reshape(1, 1)
    out = pl.pallas_call(
        functools.partial(_adj_kernel, n=n, rblk=rblk),
        grid=(n // rblk,),
        in_specs=[
            pl.BlockSpec(memory_space=pltpu.SMEM),
            pl.BlockSpec((rblk, 2 * dim), lambda g: (g, 0)),
            pl.BlockSpec((2 * dim, n), lambda g: (0, 0)),
        ],
        out_specs=pl.BlockSpec((rblk, n), lambda g: (g, 0)),
        out_shape=jax.ShapeDtypeStruct((n, n), f32),
        scratch_shapes=[
            pltpu.VMEM((rblk, n), f32),
            pltpu.VMEM((rblk, n), jnp.int32),
            pltpu.VMEM((rblk, n), jnp.int16),
        ],
    )(s10, nvcat, wbig)
    return out


# speedup vs baseline: 1.2884x; 1.1836x over previous
import functools

import jax
import jax.numpy as jnp
import numpy as np
from jax.experimental import pallas as pl
from jax.experimental.pallas import tpu as pltpu

ALPHA = 3.0
TOPK = 64


def _rotl(x, r):
    return jnp.left_shift(x, r) | jax.lax.shift_right_logical(x, 32 - r)


def _threefry_bits(p):
    ks0 = np.int32(0)
    ks1 = np.int32(42)
    ks2 = np.int32((0 ^ 42 ^ 0x1BD11BDA) & 0xFFFFFFFF)
    rot_a = (13, 15, 26, 6)
    rot_b = (17, 29, 16, 24)

    x0 = jnp.zeros_like(p) + ks0
    x1 = p + ks1

    def rounds(x0, x1, rots):
        for r in rots:
            x0 = x0 + x1
            x1 = _rotl(x1, r)
            x1 = x0 ^ x1
        return x0, x1

    x0, x1 = rounds(x0, x1, rot_a)
    x0 = x0 + ks1
    x1 = x1 + np.int32(ks2 + 1)
    x0, x1 = rounds(x0, x1, rot_b)
    x0 = x0 + ks2
    x1 = x1 + np.int32(ks0 + 2)
    x0, x1 = rounds(x0, x1, rot_a)
    x0 = x0 + ks0
    x1 = x1 + np.int32(ks1 + 3)
    x0, x1 = rounds(x0, x1, rot_b)
    x0 = x0 + ks1
    x1 = x1 + np.int32(ks2 + 4)
    x0, x1 = rounds(x0, x1, rot_a)
    x0 = x0 + ks2
    x1 = x1 + np.int32(ks0 + 5)
    return x0 ^ x1


def _embed_kernel(e1, e2, w1, w2, b1r, b2r, nvcat, *, dim):
    f32 = jnp.float32
    dot = jax.lax.dot_general
    dn_nt = (((1,), (1,)), ((), ()))

    x1 = jnp.tanh(ALPHA * (dot(e1[...], w1[...], dn_nt, preferred_element_type=f32) + b1r[...]))
    x2 = jnp.tanh(ALPHA * (dot(e2[...], w2[...], dn_nt, preferred_element_type=f32) + b2r[...]))
    nvcat[:, :dim] = x1
    nvcat[:, dim:] = x2


def _embed_t_kernel(e1, e2, w1, w2, b1c, b2c, wbig, *, dim):
    f32 = jnp.float32
    dot = jax.lax.dot_general
    dn_nt = (((1,), (1,)), ((), ()))

    x1t = jnp.tanh(ALPHA * (dot(w1[...], e1[...], dn_nt, preferred_element_type=f32) + b1c[...]))
    wbig[dim:, :] = x1t
    x2t = jnp.tanh(ALPHA * (dot(w2[...], e2[...], dn_nt, preferred_element_type=f32) + b2c[...]))
    wbig[:dim, :] = x2t


def _adj_kernel(s10_ref, nvcat, wbig, out, pert_scr, pflat_scr, *, n, rblk, nbisect):
    f32 = jnp.float32
    i32 = jnp.int32
    g = pl.program_id(0)

    @pl.when(g == 0)
    def _():
        ii = jax.lax.broadcasted_iota(i32, (rblk, n), 0)
        jj = jax.lax.broadcasted_iota(i32, (rblk, n), 1)
        pflat_scr[...] = ii * n + jj

    z = nvcat[...]
    s = 0.5 * jax.lax.dot_general(
        z, wbig[...], (((1,), (0,)), ((), ())), preferred_element_type=f32)

    adj = jnp.maximum(s, 0.0) + jnp.log1p(jnp.exp(-jnp.abs(s))) - 0.5
    out[...] = adj

    p = pflat_scr[...] + g * (rblk * n)
    bits = _threefry_bits(p)
    fbits = jax.lax.shift_right_logical(bits, 9) | np.int32(0x3F800000)
    u = jax.lax.bitcast_convert_type(fbits, f32) - 1.0
    noise = -jnp.log(-jnp.log(u + 1e-10) + 1e-10)
    pert = adj + noise
    pert_scr[...] = pert

    hi = jnp.max(pert, axis=1, keepdims=True)
    lo = jnp.full((rblk, 1), -3.7, dtype=f32)

    def body(_, carry):
        lo, hi = carry
        mid = 0.5 * (lo + hi)
        cnt = jnp.sum((pert_scr[...] >= mid).astype(f32), axis=1, keepdims=True)
        big = cnt >= float(TOPK)
        return jnp.where(big, mid, lo), jnp.where(big, hi, mid)

    lo, hi = jax.lax.fori_loop(0, nbisect, body, (lo, hi))
    thr = lo

    s10 = s10_ref[0, 0]
    mask = 1.0 / (1.0 + jnp.exp(-(pert_scr[...] - thr) * s10))
    out[...] = jnp.maximum(out[...] * mask, 0.0)


def _pick_rblk(n):
    best = 8
    for r in range(8, 257, 8):
        if n % r == 0:
            best = r
    return best


@functools.partial(jax.jit, static_argnames=())
def kernel(idx, emb1, emb2, W1, b1, W2, b2, temperature):
    n, dim = emb1.shape
    f32 = jnp.float32

    e1 = jnp.take(emb1, idx, axis=0)
    e2 = jnp.take(emb2, idx, axis=0)

    nb = n // 10 if n % 10 == 0 else n
    nvcat = pl.pallas_call(
        functools.partial(_embed_kernel, dim=dim),
        grid=(n // nb,),
        in_specs=[
            pl.BlockSpec((nb, dim), lambda g: (g, 0)),
            pl.BlockSpec((nb, dim), lambda g: (g, 0)),
            pl.BlockSpec((dim, dim), lambda g: (0, 0)),
            pl.BlockSpec((dim, dim), lambda g: (0, 0)),
            pl.BlockSpec((1, dim), lambda g: (0, 0)),
            pl.BlockSpec((1, dim), lambda g: (0, 0)),
        ],
        out_specs=pl.BlockSpec((nb, 2 * dim), lambda g: (g, 0)),
        out_shape=jax.ShapeDtypeStruct((n, 2 * dim), f32),
    )(e1, e2, W1, W2, b1.reshape(1, dim), b2.reshape(1, dim))

    wbig = pl.pallas_call(
        functools.partial(_embed_t_kernel, dim=dim),
        in_specs=[
            pl.BlockSpec((n, dim), lambda: (0, 0)),
            pl.BlockSpec((n, dim), lambda: (0, 0)),
            pl.BlockSpec((dim, dim), lambda: (0, 0)),
            pl.BlockSpec((dim, dim), lambda: (0, 0)),
            pl.BlockSpec((dim, 1), lambda: (0, 0)),
            pl.BlockSpec((dim, 1), lambda: (0, 0)),
        ],
        out_specs=pl.BlockSpec((2 * dim, n), lambda: (0, 0)),
        out_shape=jax.ShapeDtypeStruct((2 * dim, n), f32),
    )(e1, e2, W1, W2, b1.reshape(dim, 1), b2.reshape(dim, 1))

    rblk = _pick_rblk(n)
    s10 = (10.0 / temperature.astype(f32)).reshape(1, 1)
    out = pl.pallas_call(
        functools.partial(_adj_kernel, n=n, rblk=rblk, nbisect=14),
        grid=(n // rblk,),
        in_specs=[
            pl.BlockSpec(memory_space=pltpu.SMEM),
            pl.BlockSpec((rblk, 2 * dim), lambda g: (g, 0)),
            pl.BlockSpec((2 * dim, n), lambda g: (0, 0)),
        ],
        out_specs=pl.BlockSpec((rblk, n), lambda g: (g, 0)),
        out_shape=jax.ShapeDtypeStruct((n, n), f32),
        scratch_shapes=[
            pltpu.VMEM((rblk, n), f32),
            pltpu.VMEM((rblk, n), jnp.int32),
        ],
    )(s10, nvcat, wbig)
    return out


# speedup vs baseline: 1.4791x; 1.1479x over previous
import functools

import jax
import jax.numpy as jnp
import numpy as np
from jax.experimental import pallas as pl
from jax.experimental.pallas import tpu as pltpu

ALPHA = 3.0
TOPK = 64


def _rotl(x, r):
    return jnp.left_shift(x, r) | jax.lax.shift_right_logical(x, 32 - r)


def _threefry_bits(p):
    ks0 = np.int32(0)
    ks1 = np.int32(42)
    ks2 = np.int32((0 ^ 42 ^ 0x1BD11BDA) & 0xFFFFFFFF)
    rot_a = (13, 15, 26, 6)
    rot_b = (17, 29, 16, 24)

    x0 = jnp.zeros_like(p) + ks0
    x1 = p + ks1

    def rounds(x0, x1, rots):
        for r in rots:
            x0 = x0 + x1
            x1 = _rotl(x1, r)
            x1 = x0 ^ x1
        return x0, x1

    x0, x1 = rounds(x0, x1, rot_a)
    x0 = x0 + ks1
    x1 = x1 + np.int32(ks2 + 1)
    x0, x1 = rounds(x0, x1, rot_b)
    x0 = x0 + ks2
    x1 = x1 + np.int32(ks0 + 2)
    x0, x1 = rounds(x0, x1, rot_a)
    x0 = x0 + ks0
    x1 = x1 + np.int32(ks1 + 3)
    x0, x1 = rounds(x0, x1, rot_b)
    x0 = x0 + ks1
    x1 = x1 + np.int32(ks2 + 4)
    x0, x1 = rounds(x0, x1, rot_a)
    x0 = x0 + ks2
    x1 = x1 + np.int32(ks0 + 5)
    return x0 ^ x1


def _embed_kernel(e1, e2, w1, w2, b1r, b2r, nvcat, *, dim):
    f32 = jnp.float32
    dot = jax.lax.dot_general
    dn_nt = (((1,), (1,)), ((), ()))

    x1 = jnp.tanh(ALPHA * (dot(e1[...], w1[...], dn_nt, preferred_element_type=f32) + b1r[...]))
    x2 = jnp.tanh(ALPHA * (dot(e2[...], w2[...], dn_nt, preferred_element_type=f32) + b2r[...]))
    nvcat[:, :dim] = x1
    nvcat[:, dim:] = x2


def _embed_t_kernel(e1, e2, w1, w2, b1c, b2c, wbig, *, dim):
    f32 = jnp.float32
    dot = jax.lax.dot_general
    dn_nt = (((1,), (1,)), ((), ()))

    x1t = jnp.tanh(ALPHA * (dot(w1[...], e1[...], dn_nt, preferred_element_type=f32) + b1c[...]))
    wbig[dim:, :] = x1t
    x2t = jnp.tanh(ALPHA * (dot(w2[...], e2[...], dn_nt, preferred_element_type=f32) + b2c[...]))
    wbig[:dim, :] = x2t


def _adj_kernel(s10_ref, nvcat, wbig, out, pert_scr, pflat_scr, *, n, rblk, nbisect):
    f32 = jnp.float32
    i32 = jnp.int32
    g = pl.program_id(0)

    @pl.when(g == 0)
    def _():
        ii = jax.lax.broadcasted_iota(i32, (rblk, n), 0)
        jj = jax.lax.broadcasted_iota(i32, (rblk, n), 1)
        pflat_scr[...] = ii * n + jj

    z = nvcat[...]
    s = 0.5 * jax.lax.dot_general(
        z, wbig[...], (((1,), (0,)), ((), ())), preferred_element_type=f32)

    adj = jnp.maximum(s, 0.0) + jnp.log1p(jnp.exp(-jnp.abs(s))) - 0.5
    out[...] = adj

    p = pflat_scr[...] + g * (rblk * n)
    bits = _threefry_bits(p)
    fbits = jax.lax.shift_right_logical(bits, 9) | np.int32(0x3F800000)
    u = jax.lax.bitcast_convert_type(fbits, f32) - 1.0
    noise = -jnp.log(-jnp.log(u + 1e-10) + 1e-10)
    pert = adj + noise
    pert_scr[...] = pert

    hi = jnp.max(pert, axis=1, keepdims=True)
    lo = jnp.full((rblk, 1), -3.7, dtype=f32)

    def body(_, carry):
        lo, hi = carry
        mid = 0.5 * (lo + hi)
        cnt = jnp.sum((pert_scr[...] >= mid).astype(f32), axis=1, keepdims=True)
        big = cnt >= float(TOPK)
        return jnp.where(big, mid, lo), jnp.where(big, hi, mid)

    lo, hi = jax.lax.fori_loop(0, nbisect, body, (lo, hi))
    thr = lo

    s10 = s10_ref[0, 0]
    mask = 1.0 / (1.0 + jnp.exp(-(pert_scr[...] - thr) * s10))
    out[...] = jnp.maximum(out[...] * mask, 0.0)


def _pick_rblk(n):
    best = 8
    for r in range(8, 81, 8):
        if n % r == 0:
            best = r
    return best


@functools.partial(jax.jit, static_argnames=())
def kernel(idx, emb1, emb2, W1, b1, W2, b2, temperature):
    n, dim = emb1.shape
    f32 = jnp.float32

    e1 = jnp.take(emb1, idx, axis=0)
    e2 = jnp.take(emb2, idx, axis=0)

    nb = n // 10 if n % 10 == 0 else n
    nvcat = pl.pallas_call(
        functools.partial(_embed_kernel, dim=dim),
        grid=(n // nb,),
        in_specs=[
            pl.BlockSpec((nb, dim), lambda g: (g, 0)),
            pl.BlockSpec((nb, dim), lambda g: (g, 0)),
            pl.BlockSpec((dim, dim), lambda g: (0, 0)),
            pl.BlockSpec((dim, dim), lambda g: (0, 0)),
            pl.BlockSpec((1, dim), lambda g: (0, 0)),
            pl.BlockSpec((1, dim), lambda g: (0, 0)),
        ],
        out_specs=pl.BlockSpec((nb, 2 * dim), lambda g: (g, 0)),
        out_shape=jax.ShapeDtypeStruct((n, 2 * dim), f32),
    )(e1, e2, W1, W2, b1.reshape(1, dim), b2.reshape(1, dim))

    wbig = pl.pallas_call(
        functools.partial(_embed_t_kernel, dim=dim),
        in_specs=[
            pl.BlockSpec((n, dim), lambda: (0, 0)),
            pl.BlockSpec((n, dim), lambda: (0, 0)),
            pl.BlockSpec((dim, dim), lambda: (0, 0)),
            pl.BlockSpec((dim, dim), lambda: (0, 0)),
            pl.BlockSpec((dim, 1), lambda: (0, 0)),
            pl.BlockSpec((dim, 1), lambda: (0, 0)),
        ],
        out_specs=pl.BlockSpec((2 * dim, n), lambda: (0, 0)),
        out_shape=jax.ShapeDtypeStruct((2 * dim, n), f32),
    )(e1, e2, W1, W2, b1.reshape(dim, 1), b2.reshape(dim, 1))

    rblk = _pick_rblk(n)
    s10 = (10.0 / temperature.astype(f32)).reshape(1, 1)
    out = pl.pallas_call(
        functools.partial(_adj_kernel, n=n, rblk=rblk, nbisect=14),
        grid=(n // rblk,),
        in_specs=[
            pl.BlockSpec(memory_space=pltpu.SMEM),
            pl.BlockSpec((rblk, 2 * dim), lambda g: (g, 0)),
            pl.BlockSpec((2 * dim, n), lambda g: (0, 0)),
        ],
        out_specs=pl.BlockSpec((rblk, n), lambda g: (g, 0)),
        out_shape=jax.ShapeDtypeStruct((n, n), f32),
        scratch_shapes=[
            pltpu.VMEM((rblk, n), f32),
            pltpu.VMEM((rblk, n), jnp.int32),
        ],
    )(s10, nvcat, wbig)
    return out
